# Initial kernel scaffold; baseline (speedup 1.0000x reference)
import jax
import jax.numpy as jnp
from jax.experimental import pallas as pl


def kernel(loc_data, cnf_data, reg_data, targets, priors):
    raise NotImplementedError("write your pallas kernel here")


# speedup vs baseline: 39.9295x; 39.9295x over previous
import jax
import jax.numpy as jnp
from jax.experimental import pallas as pl
from jax.experimental.pallas import tpu as pltpu

_NP = 20000
_R, _C = 160, 128
_PP = _R * _C
_NCLS = 4
_NOBJ = 50
_TH = 0.5
_NEGPOS = 3
_VAR = 0.1
_ALPHA = 0.1


def _smooth_l1(d):
    ad = jnp.abs(d)
    return jnp.where(ad < 1.0, 0.5 * ad * ad, ad - 0.5)


def _body(tgt_ref, loc_ref, cnf_ref, reg_ref, pri_ref,
          out_l, out_c, out_r, out_n):
    b = pl.program_id(0)
    f32 = jnp.float32

    pcx = pri_ref[0]
    pcy = pri_ref[1]
    pw = pri_ref[2]
    ph = pri_ref[3]
    px0 = pcx - pw * 0.5
    py0 = pcy - ph * 0.5
    px1 = pcx + pw * 0.5
    py1 = pcy + ph * 0.5
    area_p = (px1 - px0) * (py1 - py0)

    row = jax.lax.broadcasted_iota(jnp.int32, (_R, _C), 0)
    col = jax.lax.broadcasted_iota(jnp.int32, (_R, _C), 1)
    idx = row * _C + col
    valid = idx < _NP

    bto = jnp.full((_R, _C), -1.0, f32)
    bti = jnp.zeros((_R, _C), jnp.int32)
    bpi = []
    for j in range(_NOBJ):
        tx0 = tgt_ref[0, j, 0]
        ty0 = tgt_ref[0, j, 1]
        tx1 = tgt_ref[0, j, 2]
        ty1 = tgt_ref[0, j, 3]
        area_t = (tx1 - tx0) * (ty1 - ty0)
        iw = jnp.maximum(jnp.minimum(tx1, px1) - jnp.maximum(tx0, px0), 0.0)
        ih = jnp.maximum(jnp.minimum(ty1, py1) - jnp.maximum(ty0, py0), 0.0)
        inter = iw * ih
        ov = inter / (area_t + area_p - inter)
        upd = ov > bto
        bto = jnp.where(upd, ov, bto)
        bti = jnp.where(upd, j, bti)
        m = jnp.max(ov)
        bpi.append(jnp.min(jnp.where(ov == m, idx, _PP)))

    for j in range(_NOBJ):
        hit = idx == bpi[j]
        bto = jnp.where(hit, 2.0, bto)
        bti = jnp.where(hit, j, bti)

    mx0 = jnp.zeros((_R, _C), f32)
    my0 = jnp.zeros((_R, _C), f32)
    mx1 = jnp.zeros((_R, _C), f32)
    my1 = jnp.zeros((_R, _C), f32)
    lbl = jnp.zeros((_R, _C), f32)
    rgt = jnp.zeros((_R, _C), f32)
    for j in range(_NOBJ):
        mj = bti == j
        mx0 = jnp.where(mj, tgt_ref[0, j, 0], mx0)
        my0 = jnp.where(mj, tgt_ref[0, j, 1], my0)
        mx1 = jnp.where(mj, tgt_ref[0, j, 2], mx1)
        my1 = jnp.where(mj, tgt_ref[0, j, 3], my1)
        lbl = jnp.where(mj, tgt_ref[0, j, 4], lbl)
        rgt = jnp.where(mj, tgt_ref[0, j, 5], rgt)

    conf = jnp.where(bto < _TH, 0, lbl.astype(jnp.int32) + 1)
    pos = conf > 0
    posf = pos.astype(f32)
    num_pos = jnp.sum(posf)

    lt0 = ((mx0 + mx1) * 0.5 - pcx) / (_VAR * pw)
    lt1 = ((my0 + my1) * 0.5 - pcy) / (_VAR * ph)
    loss_l = jnp.sum((_smooth_l1(loc_ref[0, 0] - lt0)
                      + _smooth_l1(loc_ref[0, 1] - lt1)) * posf)

    loss_r = jnp.sum(_smooth_l1(reg_ref[0, 0] - rgt) * posf)

    c0 = cnf_ref[0, 0]
    c1 = cnf_ref[0, 1]
    c2 = cnf_ref[0, 2]
    c3 = cnf_ref[0, 3]
    cm = jnp.maximum(jnp.maximum(c0, c1), jnp.maximum(c2, c3))
    lse = cm + jnp.log(jnp.exp(c0 - cm) + jnp.exp(c1 - cm)
                       + jnp.exp(c2 - cm) + jnp.exp(c3 - cm))
    g = jnp.where(conf == 0, c0, 0.0) + jnp.where(conf == 1, c1, 0.0) \
        + jnp.where(conf == 2, c2, 0.0) + jnp.where(conf == 3, c3, 0.0)
    lca = lse - g
    lca = jnp.where(pos, 0.0, lca)
    lca = jnp.where(valid, lca, -1.0)

    bits = jax.lax.bitcast_convert_type(lca, jnp.int32)
    num_neg = jnp.minimum(_NEGPOS * num_pos.astype(jnp.int32), _NP - 1)
    thr = jnp.int32(0)
    for k in range(30, -1, -1):
        cand = thr | jnp.int32(1 << k)
        cnt = jnp.sum(jnp.where(bits >= cand, 1, 0))
        thr = jnp.where(cnt >= num_neg, cand, thr)
    neg = bits >= thr
    self32 = jnp.logical_or(pos, neg).astype(f32)

    bce = jnp.zeros((_R, _C), f32)
    for k, ck in enumerate((c0, c1, c2, c3)):
        st = jnp.where(conf == k, 1.0 - _ALPHA, 0.0) + _ALPHA / 4.0
        bce = bce + (jnp.maximum(ck, 0.0) - ck * st
                     + jnp.log1p(jnp.exp(-jnp.abs(ck))))
    loss_c = jnp.sum(bce * self32)

    @pl.when(b == 0)
    def _():
        out_l[...] = jnp.zeros((8, 128), f32)
        out_c[...] = jnp.zeros((8, 128), f32)
        out_r[...] = jnp.zeros((8, 128), f32)
        out_n[...] = jnp.zeros((8, 128), f32)

    out_l[...] = out_l[...] + loss_l
    out_c[...] = out_c[...] + loss_c
    out_r[...] = out_r[...] + loss_r
    out_n[...] = out_n[...] + num_pos


def kernel(loc_data, cnf_data, reg_data, targets, priors):
    bs = loc_data.shape[0]
    pad = _PP - _NP
    f32 = jnp.float32

    pad_pri = jnp.broadcast_to(
        jnp.array([-5.0, -5.0, 1e-3, 1e-3], f32), (pad, 4))
    pri = jnp.concatenate([priors.astype(f32), pad_pri], axis=0)
    pri = pri.T.reshape(4, _R, _C)

    def pad_t(x):
        x = jnp.pad(x, ((0, 0), (0, pad), (0, 0)))
        return x.transpose(0, 2, 1).reshape(bs, x.shape[2], _R, _C)

    loc = pad_t(loc_data)
    cnf = pad_t(cnf_data)
    reg = pad_t(reg_data)
    tgt = targets.reshape(bs, _NOBJ, 6)

    out_shape = [jax.ShapeDtypeStruct((8, 128), f32)] * 4
    outs = pl.pallas_call(
        _body,
        grid=(bs,),
        in_specs=[
            pl.BlockSpec((1, _NOBJ, 6), lambda b: (b, 0, 0),
                         memory_space=pltpu.SMEM),
            pl.BlockSpec((1, 2, _R, _C), lambda b: (b, 0, 0, 0)),
            pl.BlockSpec((1, _NCLS, _R, _C), lambda b: (b, 0, 0, 0)),
            pl.BlockSpec((1, 1, _R, _C), lambda b: (b, 0, 0, 0)),
            pl.BlockSpec((4, _R, _C), lambda b: (0, 0, 0)),
        ],
        out_specs=[pl.BlockSpec((8, 128), lambda b: (0, 0))] * 4,
        out_shape=out_shape,
    )(tgt, loc, cnf, reg, pri)

    l, c, r, n = [o[0, 0] for o in outs]
    return (l / n, c / n, r / n)


# speedup vs baseline: 41.4070x; 1.0370x over previous
import jax
import jax.numpy as jnp
from jax.experimental import pallas as pl
from jax.experimental.pallas import tpu as pltpu

_NP = 20000
_R, _C = 160, 128
_PP = _R * _C
_NCLS = 4
_NOBJ = 50
_TH = 0.5
_NEGPOS = 3
_VAR = 0.1
_ALPHA = 0.1


def _smooth_l1(d):
    ad = jnp.abs(d)
    return jnp.where(ad < 1.0, 0.5 * ad * ad, ad - 0.5)


_SPP = 2


def _one_sample(tgt_ref, loc_ref, cnf_ref, reg_ref, s, pri):
    f32 = jnp.float32
    pcx, pcy, pw, ph, px0, py0, px1, py1, area_p, idx, valid = pri

    bto = jnp.full((_R, _C), -1.0, f32)
    bti = jnp.zeros((_R, _C), jnp.int32)
    bpi = []
    for j in range(_NOBJ):
        tx0 = tgt_ref[0, s, j, 0]
        ty0 = tgt_ref[0, s, j, 1]
        tx1 = tgt_ref[0, s, j, 2]
        ty1 = tgt_ref[0, s, j, 3]
        area_t = (tx1 - tx0) * (ty1 - ty0)
        iw = jnp.maximum(jnp.minimum(tx1, px1) - jnp.maximum(tx0, px0), 0.0)
        ih = jnp.maximum(jnp.minimum(ty1, py1) - jnp.maximum(ty0, py0), 0.0)
        inter = iw * ih
        ov = inter / (area_t + area_p - inter)
        upd = ov > bto
        bto = jnp.where(upd, ov, bto)
        bti = jnp.where(upd, j, bti)
        m = jnp.max(ov)
        bpi.append(jnp.min(jnp.where(ov == m, idx, _PP)))

    for j in range(_NOBJ):
        hit = idx == bpi[j]
        bto = jnp.where(hit, 2.0, bto)
        bti = jnp.where(hit, j, bti)

    mcx = jnp.zeros((_R, _C), f32)
    mcy = jnp.zeros((_R, _C), f32)
    lbl = jnp.zeros((_R, _C), f32)
    rgt = jnp.zeros((_R, _C), f32)
    for j in range(_NOBJ):
        mj = bti == j
        mcx = jnp.where(mj, (tgt_ref[0, s, j, 0] + tgt_ref[0, s, j, 2]) * 0.5, mcx)
        mcy = jnp.where(mj, (tgt_ref[0, s, j, 1] + tgt_ref[0, s, j, 3]) * 0.5, mcy)
        lbl = jnp.where(mj, tgt_ref[0, s, j, 4], lbl)
        rgt = jnp.where(mj, tgt_ref[0, s, j, 5], rgt)

    conf = jnp.where(bto < _TH, 0, lbl.astype(jnp.int32) + 1)
    pos = conf > 0
    posf = pos.astype(f32)
    num_pos = jnp.sum(posf)

    lt0 = (mcx - pcx) / (_VAR * pw)
    lt1 = (mcy - pcy) / (_VAR * ph)
    loss_l = jnp.sum((_smooth_l1(loc_ref[0, s, 0] - lt0)
                      + _smooth_l1(loc_ref[0, s, 1] - lt1)) * posf)

    loss_r = jnp.sum(_smooth_l1(reg_ref[0, s, 0] - rgt) * posf)

    c0 = cnf_ref[0, s, 0]
    c1 = cnf_ref[0, s, 1]
    c2 = cnf_ref[0, s, 2]
    c3 = cnf_ref[0, s, 3]
    cm = jnp.maximum(jnp.maximum(c0, c1), jnp.maximum(c2, c3))
    lse = cm + jnp.log(jnp.exp(c0 - cm) + jnp.exp(c1 - cm)
                       + jnp.exp(c2 - cm) + jnp.exp(c3 - cm))
    g = jnp.where(conf == 0, c0, 0.0) + jnp.where(conf == 1, c1, 0.0) \
        + jnp.where(conf == 2, c2, 0.0) + jnp.where(conf == 3, c3, 0.0)
    lca = lse - g
    lca = jnp.where(pos, 0.0, lca)
    lca = jnp.where(valid, lca, -1.0)

    bits = jax.lax.bitcast_convert_type(lca, jnp.int32)
    num_neg = jnp.minimum(_NEGPOS * num_pos.astype(jnp.int32), _NP - 1)
    thr = jnp.int32(0)
    for k in range(30, -1, -1):
        cand = thr | jnp.int32(1 << k)
        cnt = jnp.sum(jnp.where(bits >= cand, 1, 0))
        thr = jnp.where(cnt >= num_neg, cand, thr)
    neg = bits >= thr
    self32 = jnp.logical_or(pos, neg).astype(f32)

    bce = jnp.zeros((_R, _C), f32)
    for k, ck in enumerate((c0, c1, c2, c3)):
        st = jnp.where(conf == k, 1.0 - _ALPHA, 0.0) + _ALPHA / 4.0
        bce = bce + (jnp.maximum(ck, 0.0) - ck * st
                     + jnp.log1p(jnp.exp(-jnp.abs(ck))))
    loss_c = jnp.sum(bce * self32)

    return loss_l, loss_c, loss_r, num_pos


def _body(tgt_ref, loc_ref, cnf_ref, reg_ref, pri_ref,
          out_l, out_c, out_r, out_n):
    b = pl.program_id(0)
    f32 = jnp.float32

    pcx = pri_ref[0]
    pcy = pri_ref[1]
    pw = pri_ref[2]
    ph = pri_ref[3]
    px0 = pcx - pw * 0.5
    py0 = pcy - ph * 0.5
    px1 = pcx + pw * 0.5
    py1 = pcy + ph * 0.5
    area_p = (px1 - px0) * (py1 - py0)
    row = jax.lax.broadcasted_iota(jnp.int32, (_R, _C), 0)
    col = jax.lax.broadcasted_iota(jnp.int32, (_R, _C), 1)
    idx = row * _C + col
    valid = idx < _NP
    pri = (pcx, pcy, pw, ph, px0, py0, px1, py1, area_p, idx, valid)

    acc = [jnp.float32(0.0)] * 4
    for s in range(_SPP):
        res = _one_sample(tgt_ref, loc_ref, cnf_ref, reg_ref, s, pri)
        acc = [a + r for a, r in zip(acc, res)]

    @pl.when(b == 0)
    def _():
        out_l[...] = jnp.zeros((8, 128), f32)
        out_c[...] = jnp.zeros((8, 128), f32)
        out_r[...] = jnp.zeros((8, 128), f32)
        out_n[...] = jnp.zeros((8, 128), f32)

    out_l[...] = out_l[...] + acc[0]
    out_c[...] = out_c[...] + acc[1]
    out_r[...] = out_r[...] + acc[2]
    out_n[...] = out_n[...] + acc[3]


def kernel(loc_data, cnf_data, reg_data, targets, priors):
    bs = loc_data.shape[0]
    pad = _PP - _NP
    f32 = jnp.float32

    pad_pri = jnp.broadcast_to(
        jnp.array([-5.0, -5.0, 1e-3, 1e-3], f32), (pad, 4))
    pri = jnp.concatenate([priors.astype(f32), pad_pri], axis=0)
    pri = pri.T.reshape(4, _R, _C)

    def pad_t(x):
        x = jnp.pad(x, ((0, 0), (0, pad), (0, 0)))
        return x.transpose(0, 2, 1).reshape(bs, x.shape[2], _R, _C)

    loc = pad_t(loc_data)
    cnf = pad_t(cnf_data)
    reg = pad_t(reg_data)
    tgt = targets.reshape(bs, _NOBJ, 6)

    loc = loc.reshape(bs // _SPP, _SPP, 2, _R, _C)
    cnf = cnf.reshape(bs // _SPP, _SPP, _NCLS, _R, _C)
    reg = reg.reshape(bs // _SPP, _SPP, 1, _R, _C)
    tgt = tgt.reshape(bs // _SPP, _SPP, _NOBJ, 6)

    out_shape = [jax.ShapeDtypeStruct((8, 128), f32)] * 4
    outs = pl.pallas_call(
        _body,
        grid=(bs // _SPP,),
        in_specs=[
            pl.BlockSpec((1, _SPP, _NOBJ, 6), lambda b: (b, 0, 0, 0),
                         memory_space=pltpu.SMEM),
            pl.BlockSpec((1, _SPP, 2, _R, _C), lambda b: (b, 0, 0, 0, 0)),
            pl.BlockSpec((1, _SPP, _NCLS, _R, _C), lambda b: (b, 0, 0, 0, 0)),
            pl.BlockSpec((1, _SPP, 1, _R, _C), lambda b: (b, 0, 0, 0, 0)),
            pl.BlockSpec((4, _R, _C), lambda b: (0, 0, 0)),
        ],
        out_specs=[pl.BlockSpec((8, 128), lambda b: (0, 0))] * 4,
        out_shape=out_shape,
    )(tgt, loc, cnf, reg, pri)

    l, c, r, n = [o[0, 0] for o in outs]
    return (l / n, c / n, r / n)
